# Initial kernel scaffold; baseline (speedup 1.0000x reference)
import jax
import jax.numpy as jnp
from jax.experimental import pallas as pl


def kernel(x, edge_index, adj_values, W1, W2, W3):
    raise NotImplementedError("write your pallas kernel here")


# speedup vs baseline: 2.1810x; 2.1810x over previous
import functools

import jax
import jax.numpy as jnp
from jax import lax
from jax.experimental import pallas as pl
from jax.experimental.pallas import tpu as pltpu
from jax.experimental.pallas import tpu_sc as plsc

N = 10000
E = 320000
D = 128

NUM_CORES = 2
NUM_SUBCORES = 16
NUM_TILES = NUM_CORES * NUM_SUBCORES
CH = 128
EPT = 10240
EPAD = EPT * NUM_TILES
NCHUNK = EPT // CH
N_ACC = 10240
ROWS_PER_TILE = N_ACC // NUM_SUBCORES
ZR = 128


def _spmm_body(h_hbm, col_hbm, row_hbm, adj_hbm, out_hbm,
               col_v, row_v, adj_v, rows_v, zero_v, acc_sh, sem):
    cid = lax.axis_index("c")
    sid = lax.axis_index("s")
    wid = sid * NUM_CORES + cid

    def zbody(i, _):
        r = i // (D // 16)
        c = (i % (D // 16)) * 16
        zero_v[r, pl.ds(c, 16)] = jnp.zeros((16,), jnp.float32)
        return _
    lax.fori_loop(0, ZR * (D // 16), zbody, 0)
    for z in range(ROWS_PER_TILE // ZR):
        pltpu.sync_copy(zero_v, acc_sh.at[pl.ds(sid * ROWS_PER_TILE + z * ZR, ZR)])
    plsc.subcore_barrier()

    def chunk_body(i, _):
        base = wid * EPT + i * CH
        pltpu.sync_copy(col_hbm.at[pl.ds(base, CH)], col_v)
        pltpu.sync_copy(row_hbm.at[pl.ds(base, CH)], row_v)
        pltpu.sync_copy(adj_hbm.at[pl.ds(base, CH)], adj_v)
        pltpu.async_copy(h_hbm.at[col_v], rows_v, sem).wait()

        def scale_body(j, _):
            a = plsc.load_gather(adj_v, [jnp.full((16,), j, jnp.int32)])
            for r in range(D // 16):
                rows_v[j, pl.ds(r * 16, 16)] = rows_v[j, pl.ds(r * 16, 16)] * a
            return _
        lax.fori_loop(0, CH, scale_body, 0)

        pltpu.sync_copy(rows_v, acc_sh.at[row_v], add=True)
        return _
    lax.fori_loop(0, NCHUNK, chunk_body, 0)

    plsc.subcore_barrier()
    pltpu.sync_copy(acc_sh.at[pl.ds(sid * ROWS_PER_TILE, ROWS_PER_TILE)],
                    out_hbm.at[cid, pl.ds(sid * ROWS_PER_TILE, ROWS_PER_TILE)])


_spmm = functools.partial(
    pl.kernel,
    out_type=jax.ShapeDtypeStruct((NUM_CORES, N_ACC, D), jnp.float32),
    mesh=plsc.VectorSubcoreMesh(core_axis_name="c", subcore_axis_name="s"),
    compiler_params=pltpu.CompilerParams(needs_layout_passes=False),
    scratch_types=[
        pltpu.VMEM((CH,), jnp.int32),
        pltpu.VMEM((CH,), jnp.int32),
        pltpu.VMEM((CH,), jnp.float32),
        pltpu.VMEM((CH, D), jnp.float32),
        pltpu.VMEM((ZR, D), jnp.float32),
        pltpu.VMEM_SHARED((N_ACC, D), jnp.float32),
        pltpu.SemaphoreType.DMA,
    ],
)(_spmm_body)


BR = 1000


def _mm_body(x_ref, w_ref, o_ref):
    o_ref[...] = lax.dot_general(x_ref[...], w_ref[...],
                                 (((1,), (1,)), ((), ())))


def _mm(x, w):
    return pl.pallas_call(
        _mm_body,
        grid=(N // BR,),
        in_specs=[
            pl.BlockSpec((BR, D), lambda i: (i, 0)),
            pl.BlockSpec((D, D), lambda i: (0, 0)),
        ],
        out_specs=pl.BlockSpec((BR, D), lambda i: (i, 0)),
        out_shape=jax.ShapeDtypeStruct((N, D), jnp.float32),
    )(x, w)


def _relu_mm_body(p_ref, w_ref, o_ref):
    a = jnp.maximum(p_ref[0] + p_ref[1], 0.0)
    o_ref[...] = lax.dot_general(a, w_ref[...], (((1,), (1,)), ((), ())))


def _relu_mm(p, w):
    return pl.pallas_call(
        _relu_mm_body,
        grid=(N // BR,),
        in_specs=[
            pl.BlockSpec((NUM_CORES, BR, D), lambda i: (0, i, 0)),
            pl.BlockSpec((D, D), lambda i: (0, 0)),
        ],
        out_specs=pl.BlockSpec((BR, D), lambda i: (i, 0)),
        out_shape=jax.ShapeDtypeStruct((N, D), jnp.float32),
    )(p, w)


def _logsoftmax_body(p_ref, o_ref):
    a = p_ref[0] + p_ref[1]
    m = jnp.max(a, axis=1, keepdims=True)
    s = jnp.sum(jnp.exp(a - m), axis=1, keepdims=True)
    o_ref[...] = (a - m) - jnp.log(s)


def _logsoftmax(p):
    return pl.pallas_call(
        _logsoftmax_body,
        grid=(N // BR,),
        in_specs=[pl.BlockSpec((NUM_CORES, BR, D), lambda i: (0, i, 0))],
        out_specs=pl.BlockSpec((BR, D), lambda i: (i, 0)),
        out_shape=jax.ShapeDtypeStruct((N, D), jnp.float32),
    )(p)


def kernel(x, edge_index, adj_values, W1, W2, W3):
    row = edge_index[0]
    col = edge_index[1]
    pad = EPAD - E
    colp = jnp.concatenate([col, jnp.zeros((pad,), jnp.int32)])
    rowp = jnp.concatenate([row, jnp.zeros((pad,), jnp.int32)])
    adjp = jnp.concatenate([adj_values, jnp.zeros((pad,), jnp.float32)])

    h = _mm(x, W1)
    p = _spmm(h, colp, rowp, adjp)
    h = _relu_mm(p, W2)
    p = _spmm(h, colp, rowp, adjp)
    h = _relu_mm(p, W3)
    p = _spmm(h, colp, rowp, adjp)
    return _logsoftmax(p)


# speedup vs baseline: 3.3798x; 1.5496x over previous
import functools

import jax
import jax.numpy as jnp
from jax import lax
from jax.experimental import pallas as pl
from jax.experimental.pallas import tpu as pltpu
from jax.experimental.pallas import tpu_sc as plsc

N = 10000
E = 320000
D = 128

NUM_CORES = 2
NUM_SUBCORES = 16
NUM_TILES = NUM_CORES * NUM_SUBCORES
CH = 128
EPT = 10240
EPAD = EPT * NUM_TILES
NCHUNK = EPT // CH
N_ACC = 10240
ROWS_PER_TILE = N_ACC // NUM_SUBCORES
ZR = 128


def _spmm_body(h_hbm, col_hbm, row_hbm, adj_hbm, out_hbm,
               colr, rowc, adjc, rows0, rows1, acc_sh,
               sg0, sg1, ss0, ss1, si0, si1):
    cid = lax.axis_index("c")
    sid = lax.axis_index("s")
    wid = sid * NUM_CORES + cid

    pltpu.sync_copy(col_hbm.at[wid], colr)

    def zbody(i, carry):
        r = i // (D // 16)
        c = (i % (D // 16)) * 16
        rows0[r, pl.ds(c, 16)] = jnp.zeros((16,), jnp.float32)
        return carry
    lax.fori_loop(0, CH * (D // 16), zbody, 0)
    for z in range(ROWS_PER_TILE // CH):
        pltpu.sync_copy(rows0, acc_sh.at[pl.ds(sid * ROWS_PER_TILE + z * CH, CH)])
    plsc.subcore_barrier()

    def g_desc(i, buf, sem):
        return pltpu.make_async_copy(h_hbm.at[colr.at[i]], buf, sem)

    def s_desc(b, buf, sem):
        return pltpu.make_async_copy(buf, acc_sh.at[rowc.at[b]], sem)

    def r_desc(i, b, sem):
        return pltpu.make_async_copy(row_hbm.at[wid, i], rowc.at[b], sem)

    def a_desc(i, b, sem):
        return pltpu.make_async_copy(adj_hbm.at[wid, i], adjc.at[b], sem)

    def scale(buf, b):
        def scale_body(j, carry):
            a = plsc.load_gather(
                adjc, [jnp.full((16,), b, jnp.int32), jnp.full((16,), j, jnp.int32)])
            for r in range(D // 16):
                buf[j, pl.ds(r * 16, 16)] = buf[j, pl.ds(r * 16, 16)] * a
            return carry
        lax.fori_loop(0, CH, scale_body, 0)

    bufs = ((rows0, sg0, ss0, si0), (rows1, sg1, ss1, si1))

    r_desc(0, 0, si0).start()
    a_desc(0, 0, si0).start()
    g_desc(0, rows0, sg0).start()

    def body2(it, carry):
        for k in range(2):
            i = it * 2 + k
            buf, sg, ss, si = bufs[k]
            obuf, osg, oss, osi = bufs[1 - k]

            @pl.when(i + 1 < NCHUNK)
            def _prefetch():
                @pl.when(i > 0)
                def _drain():
                    s_desc(1 - k, obuf, oss).wait()
                r_desc(i + 1, 1 - k, osi).start()
                a_desc(i + 1, 1 - k, osi).start()
                g_desc(i + 1, obuf, osg).start()

            g_desc(i, buf, sg).wait()
            r_desc(i, k, si).wait()
            a_desc(i, k, si).wait()
            scale(buf, k)
            s_desc(k, buf, ss).start(add=True)
        return carry
    lax.fori_loop(0, NCHUNK // 2, body2, 0)
    s_desc(0, rows0, ss0).wait()
    s_desc(1, rows1, ss1).wait()

    plsc.subcore_barrier()
    pltpu.sync_copy(acc_sh.at[pl.ds(sid * ROWS_PER_TILE, ROWS_PER_TILE)],
                    out_hbm.at[cid, pl.ds(sid * ROWS_PER_TILE, ROWS_PER_TILE)])


_spmm = functools.partial(
    pl.kernel,
    out_type=jax.ShapeDtypeStruct((NUM_CORES, N_ACC, D), jnp.float32),
    mesh=plsc.VectorSubcoreMesh(core_axis_name="c", subcore_axis_name="s"),
    compiler_params=pltpu.CompilerParams(needs_layout_passes=False),
    scratch_types=[
        pltpu.VMEM((NCHUNK, CH), jnp.int32),
        pltpu.VMEM((2, CH), jnp.int32),
        pltpu.VMEM((2, CH), jnp.float32),
        pltpu.VMEM((CH, D), jnp.float32),
        pltpu.VMEM((CH, D), jnp.float32),
        pltpu.VMEM_SHARED((N_ACC, D), jnp.float32),
        pltpu.SemaphoreType.DMA,
        pltpu.SemaphoreType.DMA,
        pltpu.SemaphoreType.DMA,
        pltpu.SemaphoreType.DMA,
        pltpu.SemaphoreType.DMA,
        pltpu.SemaphoreType.DMA,
    ],
)(_spmm_body)


BR = 1000


def _mm_body(x_ref, w_ref, o_ref):
    o_ref[...] = lax.dot_general(x_ref[...], w_ref[...],
                                 (((1,), (1,)), ((), ())))


def _mm(x, w):
    return pl.pallas_call(
        _mm_body,
        grid=(N // BR,),
        in_specs=[
            pl.BlockSpec((BR, D), lambda i: (i, 0)),
            pl.BlockSpec((D, D), lambda i: (0, 0)),
        ],
        out_specs=pl.BlockSpec((BR, D), lambda i: (i, 0)),
        out_shape=jax.ShapeDtypeStruct((N, D), jnp.float32),
    )(x, w)


def _relu_mm_body(p_ref, w_ref, o_ref):
    a = jnp.maximum(p_ref[0] + p_ref[1], 0.0)
    o_ref[...] = lax.dot_general(a, w_ref[...], (((1,), (1,)), ((), ())))


def _relu_mm(p, w):
    return pl.pallas_call(
        _relu_mm_body,
        grid=(N // BR,),
        in_specs=[
            pl.BlockSpec((NUM_CORES, BR, D), lambda i: (0, i, 0)),
            pl.BlockSpec((D, D), lambda i: (0, 0)),
        ],
        out_specs=pl.BlockSpec((BR, D), lambda i: (i, 0)),
        out_shape=jax.ShapeDtypeStruct((N, D), jnp.float32),
    )(p, w)


def _logsoftmax_body(p_ref, o_ref):
    a = p_ref[0] + p_ref[1]
    m = jnp.max(a, axis=1, keepdims=True)
    s = jnp.sum(jnp.exp(a - m), axis=1, keepdims=True)
    o_ref[...] = (a - m) - jnp.log(s)


def _logsoftmax(p):
    return pl.pallas_call(
        _logsoftmax_body,
        grid=(N // BR,),
        in_specs=[pl.BlockSpec((NUM_CORES, BR, D), lambda i: (0, i, 0))],
        out_specs=pl.BlockSpec((BR, D), lambda i: (i, 0)),
        out_shape=jax.ShapeDtypeStruct((N, D), jnp.float32),
    )(p)


def kernel(x, edge_index, adj_values, W1, W2, W3):
    row = edge_index[0]
    col = edge_index[1]
    pad = EPAD - E
    shape3 = (NUM_TILES, NCHUNK, CH)
    colp = jnp.concatenate([col, jnp.zeros((pad,), jnp.int32)]).reshape(shape3)
    rowp = jnp.concatenate([row, jnp.zeros((pad,), jnp.int32)]).reshape(shape3)
    adjp = jnp.concatenate(
        [adj_values, jnp.zeros((pad,), jnp.float32)]).reshape(shape3)

    h = _mm(x, W1)
    p = _spmm(h, colp, rowp, adjp)
    h = _relu_mm(p, W2)
    p = _spmm(h, colp, rowp, adjp)
    h = _relu_mm(p, W3)
    p = _spmm(h, colp, rowp, adjp)
    return _logsoftmax(p)


# speedup vs baseline: 8.1395x; 2.4083x over previous
import functools

import jax
import jax.numpy as jnp
from jax import lax
from jax.experimental import pallas as pl
from jax.experimental.pallas import tpu as pltpu
from jax.experimental.pallas import tpu_sc as plsc

N = 10000
E = 320000
D = 128

NUM_CORES = 2
NUM_SUBCORES = 16
NUM_TILES = NUM_CORES * NUM_SUBCORES
CH = 128
EPT = 10240
EPAD = EPT * NUM_TILES
NCHUNK = EPT // CH
N_ACC = 10240
ROWS_PER_TILE = N_ACC // NUM_SUBCORES
ZR = 128


def _spmm_body(h_hbm, col_hbm, row_hbm, adj_hbm, out_hbm,
               colr, rowc, adjc, rows0, rows1, acc_sh,
               sg0, sg1, ss0, ss1, si0, si1):
    cid = lax.axis_index("c")
    sid = lax.axis_index("s")
    wid = sid * NUM_CORES + cid

    pltpu.sync_copy(col_hbm.at[wid], colr)

    def zbody(i, carry):
        r = i // (D // 16)
        c = (i % (D // 16)) * 16
        rows0[r, pl.ds(c, 16)] = jnp.zeros((16,), jnp.float32)
        return carry
    lax.fori_loop(0, CH * (D // 16), zbody, 0)
    for z in range(ROWS_PER_TILE // CH):
        pltpu.sync_copy(rows0, acc_sh.at[pl.ds(sid * ROWS_PER_TILE + z * CH, CH)])
    plsc.subcore_barrier()

    def g_desc(i, buf, sem):
        return pltpu.make_async_copy(h_hbm.at[colr.at[i]], buf, sem)

    def s_desc(b, buf, sem):
        return pltpu.make_async_copy(buf, acc_sh.at[rowc.at[b]], sem)

    def r_desc(i, b, sem):
        return pltpu.make_async_copy(row_hbm.at[wid, i], rowc.at[b], sem)

    def a_desc(i, b, sem):
        return pltpu.make_async_copy(adj_hbm.at[wid, i], adjc.at[b], sem)

    def scale(buf, b):
        def scale_body(j, carry):
            a = plsc.load_gather(
                adjc, [jnp.full((16,), b, jnp.int32), jnp.full((16,), j, jnp.int32)])
            for r in range(D // 16):
                buf[j, pl.ds(r * 16, 16)] = buf[j, pl.ds(r * 16, 16)] * a
            return carry
        lax.fori_loop(0, CH, scale_body, 0)

    bufs = ((rows0, sg0, ss0, si0), (rows1, sg1, ss1, si1))

    r_desc(0, 0, si0).start()
    a_desc(0, 0, si0).start()
    g_desc(0, rows0, sg0).start()

    def body2(it, carry):
        for k in range(2):
            i = it * 2 + k
            buf, sg, ss, si = bufs[k]
            obuf, osg, oss, osi = bufs[1 - k]

            @pl.when(i + 1 < NCHUNK)
            def _prefetch():
                @pl.when(i > 0)
                def _drain():
                    s_desc(1 - k, obuf, oss).wait()
                r_desc(i + 1, 1 - k, osi).start()
                a_desc(i + 1, 1 - k, osi).start()
                g_desc(i + 1, obuf, osg).start()

            g_desc(i, buf, sg).wait()
            r_desc(i, k, si).wait()
            a_desc(i, k, si).wait()
            scale(buf, k)
            s_desc(k, buf, ss).start(add=True)
        return carry
    lax.fori_loop(0, NCHUNK // 2, body2, 0)
    s_desc(0, rows0, ss0).wait()
    s_desc(1, rows1, ss1).wait()

    plsc.subcore_barrier()
    pltpu.sync_copy(acc_sh.at[pl.ds(sid * ROWS_PER_TILE, ROWS_PER_TILE)],
                    out_hbm.at[cid, pl.ds(sid * ROWS_PER_TILE, ROWS_PER_TILE)])


_spmm = functools.partial(
    pl.kernel,
    out_type=jax.ShapeDtypeStruct((NUM_CORES, N_ACC, D), jnp.float32),
    mesh=plsc.VectorSubcoreMesh(core_axis_name="c", subcore_axis_name="s"),
    compiler_params=pltpu.CompilerParams(needs_layout_passes=False),
    scratch_types=[
        pltpu.VMEM((NCHUNK, CH), jnp.int32),
        pltpu.VMEM((2, CH), jnp.int32),
        pltpu.VMEM((2, CH), jnp.float32),
        pltpu.VMEM((CH, D), jnp.float32),
        pltpu.VMEM((CH, D), jnp.float32),
        pltpu.VMEM_SHARED((N_ACC, D), jnp.float32),
        pltpu.SemaphoreType.DMA,
        pltpu.SemaphoreType.DMA,
        pltpu.SemaphoreType.DMA,
        pltpu.SemaphoreType.DMA,
        pltpu.SemaphoreType.DMA,
        pltpu.SemaphoreType.DMA,
    ],
)(_spmm_body)


BR = 1000


def _mm_body(x_ref, w_ref, o_ref):
    o_ref[...] = lax.dot_general(x_ref[...], w_ref[...],
                                 (((1,), (1,)), ((), ())))


def _mm(x, w):
    return pl.pallas_call(
        _mm_body,
        grid=(N // BR,),
        in_specs=[
            pl.BlockSpec((BR, D), lambda i: (i, 0)),
            pl.BlockSpec((D, D), lambda i: (0, 0)),
        ],
        out_specs=pl.BlockSpec((BR, D), lambda i: (i, 0)),
        out_shape=jax.ShapeDtypeStruct((N, D), jnp.float32),
    )(x, w)


def _relu_mm_body(p_ref, w_ref, o_ref):
    a = jnp.maximum(p_ref[0] + p_ref[1], 0.0)
    o_ref[...] = lax.dot_general(a, w_ref[...], (((1,), (1,)), ((), ())))


def _relu_mm(p, w):
    return pl.pallas_call(
        _relu_mm_body,
        grid=(N // BR,),
        in_specs=[
            pl.BlockSpec((NUM_CORES, BR, D), lambda i: (0, i, 0)),
            pl.BlockSpec((D, D), lambda i: (0, 0)),
        ],
        out_specs=pl.BlockSpec((BR, D), lambda i: (i, 0)),
        out_shape=jax.ShapeDtypeStruct((N, D), jnp.float32),
    )(p, w)


def _logsoftmax_body(p_ref, o_ref):
    a = p_ref[0] + p_ref[1]
    m = jnp.max(a, axis=1, keepdims=True)
    s = jnp.sum(jnp.exp(a - m), axis=1, keepdims=True)
    o_ref[...] = (a - m) - jnp.log(s)


def _logsoftmax(p):
    return pl.pallas_call(
        _logsoftmax_body,
        grid=(N // BR,),
        in_specs=[pl.BlockSpec((NUM_CORES, BR, D), lambda i: (0, i, 0))],
        out_specs=pl.BlockSpec((BR, D), lambda i: (i, 0)),
        out_shape=jax.ShapeDtypeStruct((N, D), jnp.float32),
    )(p)


def kernel(x, edge_index, adj_values, W1, W2, W3):
    row = edge_index[0]
    col = edge_index[1]
    rpt = E // NUM_TILES
    ppt = EPT - rpt
    padc = (jnp.arange(NUM_TILES * ppt, dtype=jnp.int32) % N
            ).reshape(NUM_TILES, ppt)
    padr = (jnp.arange(NUM_TILES * ppt, dtype=jnp.int32) % N_ACC
            ).reshape(NUM_TILES, ppt)
    shape3 = (NUM_TILES, NCHUNK, CH)
    colp = jnp.concatenate(
        [col.reshape(NUM_TILES, rpt), padc], axis=1).reshape(shape3)
    rowp = jnp.concatenate(
        [row.reshape(NUM_TILES, rpt), padr], axis=1).reshape(shape3)
    adjp = jnp.concatenate(
        [adj_values.reshape(NUM_TILES, rpt),
         jnp.zeros((NUM_TILES, ppt), jnp.float32)], axis=1).reshape(shape3)

    h = _mm(x, W1)
    p = _spmm(h, colp, rowp, adjp)
    h = _relu_mm(p, W2)
    p = _spmm(h, colp, rowp, adjp)
    h = _relu_mm(p, W3)
    p = _spmm(h, colp, rowp, adjp)
    return _logsoftmax(p)


# speedup vs baseline: 9.8718x; 1.2128x over previous
import functools

import jax
import jax.numpy as jnp
from jax import lax
from jax.experimental import pallas as pl
from jax.experimental.pallas import tpu as pltpu
from jax.experimental.pallas import tpu_sc as plsc

N = 10000
E = 320000
D = 128

NUM_CORES = 2
NUM_SUBCORES = 16
NUM_TILES = NUM_CORES * NUM_SUBCORES
CH = 128
EPT = 10240
EPAD = EPT * NUM_TILES
NCHUNK = EPT // CH
N_ACC = 10240
ROWS_PER_TILE = N_ACC // NUM_SUBCORES
ZR = 128


def _spmm_body(h_hbm, col_hbm, row_hbm, adj_hbm, out_hbm,
               colr, rowc, adjc, rows0, rows1, acc_sh,
               sg0, sg1, ss0, ss1, si0, si1):
    cid = lax.axis_index("c")
    sid = lax.axis_index("s")
    wid = sid * NUM_CORES + cid

    pltpu.sync_copy(col_hbm.at[wid], colr)

    def zbody(i, carry):
        r = i // (D // 16)
        c = (i % (D // 16)) * 16
        rows0[r, pl.ds(c, 16)] = jnp.zeros((16,), jnp.float32)
        return carry
    lax.fori_loop(0, CH * (D // 16), zbody, 0)
    for z in range(ROWS_PER_TILE // CH):
        pltpu.sync_copy(rows0, acc_sh.at[pl.ds(sid * ROWS_PER_TILE + z * CH, CH)])
    plsc.subcore_barrier()

    def g_desc(i, buf, sem):
        return pltpu.make_async_copy(h_hbm.at[colr.at[i]], buf, sem)

    def s_desc(b, buf, sem):
        return pltpu.make_async_copy(buf, acc_sh.at[rowc.at[b]], sem)

    def r_desc(i, b, sem):
        return pltpu.make_async_copy(row_hbm.at[wid, i], rowc.at[b], sem)

    def a_desc(i, b, sem):
        return pltpu.make_async_copy(adj_hbm.at[wid, i], adjc.at[b], sem)

    UNROLL = 4

    def scale(buf, b):
        def scale_body(j4, carry):
            j = j4 * UNROLL
            avs = [plsc.load_gather(
                adjc, [jnp.full((16,), b, jnp.int32),
                       jnp.full((16,), j + u, jnp.int32)])
                   for u in range(UNROLL)]
            for u in range(UNROLL):
                for r in range(D // 16):
                    buf[j + u, pl.ds(r * 16, 16)] = (
                        buf[j + u, pl.ds(r * 16, 16)] * avs[u])
            return carry
        lax.fori_loop(0, CH // UNROLL, scale_body, 0)

    bufs = ((rows0, sg0, ss0, si0), (rows1, sg1, ss1, si1))

    r_desc(0, 0, si0).start()
    a_desc(0, 0, si0).start()
    g_desc(0, rows0, sg0).start()

    def body2(it, carry):
        for k in range(2):
            i = it * 2 + k
            buf, sg, ss, si = bufs[k]
            obuf, osg, oss, osi = bufs[1 - k]

            @pl.when(i + 1 < NCHUNK)
            def _prefetch():
                @pl.when(i > 0)
                def _drain():
                    s_desc(1 - k, obuf, oss).wait()
                r_desc(i + 1, 1 - k, osi).start()
                a_desc(i + 1, 1 - k, osi).start()
                g_desc(i + 1, obuf, osg).start()

            g_desc(i, buf, sg).wait()
            r_desc(i, k, si).wait()
            a_desc(i, k, si).wait()
            scale(buf, k)
            s_desc(k, buf, ss).start(add=True)
        return carry
    lax.fori_loop(0, NCHUNK // 2, body2, 0)
    s_desc(0, rows0, ss0).wait()
    s_desc(1, rows1, ss1).wait()

    plsc.subcore_barrier()
    pltpu.sync_copy(acc_sh.at[pl.ds(sid * ROWS_PER_TILE, ROWS_PER_TILE)],
                    out_hbm.at[cid, pl.ds(sid * ROWS_PER_TILE, ROWS_PER_TILE)])


_spmm = functools.partial(
    pl.kernel,
    out_type=jax.ShapeDtypeStruct((NUM_CORES, N_ACC, D), jnp.float32),
    mesh=plsc.VectorSubcoreMesh(core_axis_name="c", subcore_axis_name="s"),
    compiler_params=pltpu.CompilerParams(needs_layout_passes=False),
    scratch_types=[
        pltpu.VMEM((NCHUNK, CH), jnp.int32),
        pltpu.VMEM((2, CH), jnp.int32),
        pltpu.VMEM((2, CH), jnp.float32),
        pltpu.VMEM((CH, D), jnp.float32),
        pltpu.VMEM((CH, D), jnp.float32),
        pltpu.VMEM_SHARED((N_ACC, D), jnp.float32),
        pltpu.SemaphoreType.DMA,
        pltpu.SemaphoreType.DMA,
        pltpu.SemaphoreType.DMA,
        pltpu.SemaphoreType.DMA,
        pltpu.SemaphoreType.DMA,
        pltpu.SemaphoreType.DMA,
    ],
)(_spmm_body)


BR = 1000


def _mm_body(x_ref, w_ref, o_ref):
    o_ref[...] = lax.dot_general(x_ref[...], w_ref[...],
                                 (((1,), (1,)), ((), ())))


def _mm(x, w):
    return pl.pallas_call(
        _mm_body,
        grid=(N // BR,),
        in_specs=[
            pl.BlockSpec((BR, D), lambda i: (i, 0)),
            pl.BlockSpec((D, D), lambda i: (0, 0)),
        ],
        out_specs=pl.BlockSpec((BR, D), lambda i: (i, 0)),
        out_shape=jax.ShapeDtypeStruct((N, D), jnp.float32),
    )(x, w)


def _relu_mm_body(p_ref, w_ref, o_ref):
    a = jnp.maximum(p_ref[0] + p_ref[1], 0.0)
    o_ref[...] = lax.dot_general(a, w_ref[...], (((1,), (1,)), ((), ())))


def _relu_mm(p, w):
    return pl.pallas_call(
        _relu_mm_body,
        grid=(N // BR,),
        in_specs=[
            pl.BlockSpec((NUM_CORES, BR, D), lambda i: (0, i, 0)),
            pl.BlockSpec((D, D), lambda i: (0, 0)),
        ],
        out_specs=pl.BlockSpec((BR, D), lambda i: (i, 0)),
        out_shape=jax.ShapeDtypeStruct((N, D), jnp.float32),
    )(p, w)


def _logsoftmax_body(p_ref, o_ref):
    a = p_ref[0] + p_ref[1]
    m = jnp.max(a, axis=1, keepdims=True)
    s = jnp.sum(jnp.exp(a - m), axis=1, keepdims=True)
    o_ref[...] = (a - m) - jnp.log(s)


def _logsoftmax(p):
    return pl.pallas_call(
        _logsoftmax_body,
        grid=(N // BR,),
        in_specs=[pl.BlockSpec((NUM_CORES, BR, D), lambda i: (0, i, 0))],
        out_specs=pl.BlockSpec((BR, D), lambda i: (i, 0)),
        out_shape=jax.ShapeDtypeStruct((N, D), jnp.float32),
    )(p)


def kernel(x, edge_index, adj_values, W1, W2, W3):
    row = edge_index[0]
    col = edge_index[1]
    rpt = E // NUM_TILES
    ppt = EPT - rpt
    padc = (jnp.arange(NUM_TILES * ppt, dtype=jnp.int32) % N
            ).reshape(NUM_TILES, ppt)
    padr = (jnp.arange(NUM_TILES * ppt, dtype=jnp.int32) % N_ACC
            ).reshape(NUM_TILES, ppt)
    shape3 = (NUM_TILES, NCHUNK, CH)
    colp = jnp.concatenate(
        [col.reshape(NUM_TILES, rpt), padc], axis=1).reshape(shape3)
    rowp = jnp.concatenate(
        [row.reshape(NUM_TILES, rpt), padr], axis=1).reshape(shape3)
    adjp = jnp.concatenate(
        [adj_values.reshape(NUM_TILES, rpt),
         jnp.zeros((NUM_TILES, ppt), jnp.float32)], axis=1).reshape(shape3)

    h = _mm(x, W1)
    p = _spmm(h, colp, rowp, adjp)
    h = _relu_mm(p, W2)
    p = _spmm(h, colp, rowp, adjp)
    h = _relu_mm(p, W3)
    p = _spmm(h, colp, rowp, adjp)
    return _logsoftmax(p)
